# baseline (speedup 1.0000x reference)
import functools

import jax
import jax.numpy as jnp
import numpy as np
from jax import lax
from jax.experimental import pallas as pl
from jax.experimental.pallas import tpu as pltpu
from jax.experimental.pallas import tpu_sc as plsc

N, LQ, C = 4, 5440, 256
M, L, P, D = 8, 4, 4, 32
HW_LIST = [(64, 64), (32, 32), (16, 16), (8, 8)]
SIZES = [h * w for h, w in HW_LIST]
STARTS = [0, 4096, 5120, 5376]
V = 5440
NROW = N * LQ
QBLK = 320
GJ = LQ // QBLK
TABLE_ROWS = N * M * V

NC, NS, LANES = 2, 16, 16
NW = NC * NS
QSLICE = LQ // (NW // N)
QB = 4
ROWS_PER_CHUNK = QB * 4 * 128
NCHUNK = QSLICE // QB


def _np_consts():
    lvl = np.zeros(128, np.int32)
    for m in range(M):
        for l in range(L):
            for p in range(P):
                lvl[m * 16 + l * 4 + p] = l
    wl = np.array([HW_LIST[l][1] for l in lvl], np.float32)
    hl = np.array([HW_LIST[l][0] for l in lvl], np.float32)
    base = np.array(
        [(c // 16) * V + STARTS[lvl[c]] for c in range(128)], np.float32
    )
    sx = np.zeros((4, 128), np.float32)
    sy = np.zeros((4, 128), np.float32)
    for col in range(128):
        sx[lvl[col], col] = wl[col]
        sy[lvl[col], col] = hl[col]
    bsum = np.zeros((128, 8), np.float32)
    for col in range(128):
        bsum[col, col // 16] = 1.0
    return wl, hl, base, sx, sy, bsum


def _stage_a_body(q_ref, x_ref, rpx_ref, rpy_ref, wv_ref, bv_ref, wo_ref,
                  bo_ref, wa_ref, ba_ref, sx_ref, sy_ref, base_ref, wl_ref,
                  hl_ref, bs_ref, bst_ref,
                  val_ref, i0_ref, i1_ref,
                  w0_ref, w1_ref, w2_ref, w3_ref):
    f32 = jnp.float32
    q = q_ref[...]
    val_ref[...] = (
        jnp.dot(x_ref[...], wv_ref[...], preferred_element_type=f32,
                precision=lax.Precision.HIGHEST)
        + bv_ref[...]
    ).astype(jnp.bfloat16)
    off = jnp.dot(q, wo_ref[...], preferred_element_type=f32,
                precision=lax.Precision.HIGHEST) + bo_ref[...]
    lg = jnp.dot(q, wa_ref[...], preferred_element_type=f32,
                precision=lax.Precision.HIGHEST) + ba_ref[...]
    e = jnp.exp(lg)
    s = jnp.dot(e, bs_ref[...], preferred_element_type=f32,
                precision=lax.Precision.HIGHEST)
    rb = jnp.dot(1.0 / s, bst_ref[...], preferred_element_type=f32,
                precision=lax.Precision.HIGHEST)
    aw = e * rb
    offx = off[:, :128]
    offy = off[:, 128:]
    ix = jnp.dot(rpx_ref[...], sx_ref[...], preferred_element_type=f32,
                precision=lax.Precision.HIGHEST) \
        + offx - 0.5
    iy = jnp.dot(rpy_ref[...], sy_ref[...], preferred_element_type=f32,
                precision=lax.Precision.HIGHEST) \
        + offy - 0.5
    x0 = jnp.floor(ix)
    y0 = jnp.floor(iy)
    x1 = x0 + 1.0
    y1 = y0 + 1.0
    wx1 = ix - x0
    wx0 = 1.0 - wx1
    wy1 = iy - y0
    wy0 = 1.0 - wy1
    wl = wl_ref[...]
    hl = hl_ref[...]
    wm1 = wl - 1.0
    hm1 = hl - 1.0
    vx0 = ((x0 >= 0.0) & (x0 <= wm1)).astype(f32)
    vx1 = ((x1 >= 0.0) & (x1 <= wm1)).astype(f32)
    vy0 = ((y0 >= 0.0) & (y0 <= hm1)).astype(f32)
    vy1 = ((y1 >= 0.0) & (y1 <= hm1)).astype(f32)
    xs = jnp.clip(x0, 0.0, wm1)
    y0c = jnp.clip(y0, 0.0, hm1)
    y1c = jnp.clip(y1, 0.0, hm1)
    wleft = jnp.where(x0 < 0.0, wx1 * vx1, wx0 * vx0)
    wright = jnp.where(x0 >= 0.0, wx1 * vx1, 0.0)
    nb = pl.program_id(0).astype(f32)
    base = base_ref[...] + nb * jnp.float32(M * V)
    i0_ref[...] = (base + y0c * wl + xs).astype(jnp.int32)
    i1_ref[...] = (base + y1c * wl + xs).astype(jnp.int32)
    w0_ref[...] = aw * wy0 * vy0 * wleft
    w1_ref[...] = aw * wy0 * vy0 * wright
    w2_ref[...] = aw * wy1 * vy1 * wleft
    w3_ref[...] = aw * wy1 * vy1 * wright


def _run_stage_a(q2, x2, rpx, rpy, w_val, b_val, w_offp, b_offp, w_attnp,
                 b_attnp, sx, sy, basev, wlv, hlv, bsum, bsum_t,
                 interpret=False):
    f32 = jnp.float32
    row_spec = lambda shp: pl.BlockSpec(
        (QBLK, shp), lambda n, j: (n * GJ + j, 0))
    full_spec = lambda a, b: pl.BlockSpec((a, b), lambda n, j: (0, 0))
    outs = [jax.ShapeDtypeStruct((NROW, 256), jnp.bfloat16)]
    outs += [jax.ShapeDtypeStruct((NROW, 128), jnp.int32)] * 2
    outs += [jax.ShapeDtypeStruct((NROW, 128), f32)] * 4
    return pl.pallas_call(
        _stage_a_body,
        grid=(N, GJ),
        in_specs=[
            row_spec(256), row_spec(256), row_spec(4), row_spec(4),
            full_spec(256, 256), full_spec(1, 256),
            full_spec(256, 256), full_spec(1, 256),
            full_spec(256, 128), full_spec(1, 128),
            full_spec(4, 128), full_spec(4, 128),
            full_spec(1, 128), full_spec(1, 128), full_spec(1, 128),
            full_spec(128, 8), full_spec(8, 128),
        ],
        out_specs=[row_spec(256)] + [row_spec(128)] * 6,
        out_shape=outs,
        interpret=interpret,
    )(q2, x2, rpx, rpy, w_val, b_val, w_offp, b_offp, w_attnp, b_attnp,
      sx, sy, basev, wlv, hlv, bsum, bsum_t)


def _matmul_body(x_ref, w_ref, b_ref, o_ref):
    o_ref[...] = (
        jnp.dot(x_ref[...], w_ref[...], preferred_element_type=jnp.float32,
                precision=lax.Precision.HIGHEST)
        + b_ref[...]
    )


def _run_stage_c(x2, w_out, b_out, interpret=False):
    return pl.pallas_call(
        _matmul_body,
        grid=(N, GJ),
        in_specs=[
            pl.BlockSpec((QBLK, 256), lambda n, j: (n * GJ + j, 0)),
            pl.BlockSpec((256, 256), lambda n, j: (0, 0)),
            pl.BlockSpec((1, 256), lambda n, j: (0, 0)),
        ],
        out_specs=pl.BlockSpec((QBLK, 256), lambda n, j: (n * GJ + j, 0)),
        out_shape=jax.ShapeDtypeStruct((NROW, 256), jnp.float32),
        interpret=interpret,
    )(x2, w_out, b_out)


def _sc_body(table_ref, i0_ref, i1_ref,
             w0_ref, w1_ref, w2_ref, w3_ref, out_ref,
             idx_v, w_v, rows_v, out_v, g0, g1, iw0, iw1):
    g_sems = [g0, g1]
    iw_sems = [iw0, iw1]
    f32 = jnp.float32
    cid = lax.axis_index("c")
    sid = lax.axis_index("s")
    wid = sid * NC + cid
    n = wid // (NW // N)
    s = wid % (NW // N)
    q0 = s * QSLICE
    irefs = [i0_ref, i1_ref]
    wrefs = [w0_ref, w1_ref, w2_ref, w3_ref]

    def row0_of(tc):
        return n * LQ + q0 + tc * QB

    def issue_iw(tc, buf):
        r0 = row0_of(tc)
        for k in range(2):
            pltpu.async_copy(
                irefs[k].at[pl.ds(r0, QB)], idx_v.at[buf, k], iw_sems[buf])
        for c in range(4):
            pltpu.async_copy(
                wrefs[c].at[pl.ds(r0, QB)], w_v.at[buf, c], iw_sems[buf])

    def wait_iw(buf):
        for k in range(2):
            pltpu.make_async_copy(
                irefs[k].at[pl.ds(0, QB)], idx_v.at[buf, k],
                iw_sems[buf]).wait()
        for c in range(4):
            pltpu.make_async_copy(
                wrefs[c].at[pl.ds(0, QB)], w_v.at[buf, c],
                iw_sems[buf]).wait()

    def issue_gathers(buf):
        for k in range(2):
            for q in range(QB):
                pltpu.async_copy(
                    table_ref.at[idx_v.at[buf, k, q]],
                    rows_v.at[buf, pl.ds((k * QB + q) * 128, 128)],
                    g_sems[buf])

    def wait_gathers(buf):
        for k in range(2):
            for q in range(QB):
                pltpu.make_async_copy(
                    table_ref.at[idx_v.at[buf, k, q]],
                    rows_v.at[buf, pl.ds((k * QB + q) * 128, 128)],
                    g_sems[buf]).wait()

    def mac(tc, buf):
        def q_body(q, carry2):
            def m_body(m, carry3):
                p0 = [jnp.zeros((LANES,), f32) for _ in range(4)]
                p1 = [jnp.zeros((LANES,), f32) for _ in range(4)]
                w16s = [w_v[buf, c, q, pl.ds(m * 16, 16)] for c in range(4)]
                for r in range(16):
                    sel = jnp.full((LANES,), r, jnp.int32)
                    for c in range(4):
                        wb = w16s[c].at[sel].get(mode="promise_in_bounds")
                        row = ((c // 2) * QB + q) * 128 + m * 16 + r
                        ra, rb = plsc.unpack(
                            rows_v[buf, row, pl.ds((c % 2) * 32, 32)],
                            format=plsc.PackFormat.INTERLEAVED)
                        p0[c] = p0[c] + wb * ra
                        p1[c] = p1[c] + wb * rb
                out_v[q, pl.ds(m * D, 16)] = (p0[0] + p0[1]) + (p0[2] + p0[3])
                out_v[q, pl.ds(m * D + 16, 16)] = (
                    (p1[0] + p1[1]) + (p1[2] + p1[3]))
                return carry3

            lax.fori_loop(0, M, m_body, 0)
            return carry2

        lax.fori_loop(0, QB, q_body, 0)
        pltpu.sync_copy(out_v, out_ref.at[pl.ds(row0_of(tc), QB)])

    def half(t, buf):
        nbuf = 1 - buf
        wait_gathers(buf)
        wait_iw(nbuf)
        issue_gathers(nbuf)
        mac(t, buf)
        issue_iw(jnp.minimum(t + 2, NCHUNK - 1), buf)

    r0 = row0_of(0)
    for k in range(2):
        pltpu.sync_copy(irefs[k].at[pl.ds(r0, QB)], idx_v.at[0, k])
    for c in range(4):
        pltpu.sync_copy(wrefs[c].at[pl.ds(r0, QB)], w_v.at[0, c])
    issue_gathers(0)
    issue_iw(jnp.int32(1), 1)

    def pair_body(i, carry):
        t0 = i * 2
        half(t0, 0)
        half(t0 + 1, 1)
        return carry

    lax.fori_loop(0, NCHUNK // 2, pair_body, 0)
    wait_gathers(0)
    wait_iw(1)


def _run_stage_b(table, idxs, wts):
    mesh = plsc.VectorSubcoreMesh(core_axis_name="c", subcore_axis_name="s")
    fn = pl.kernel(
        _sc_body,
        out_type=jax.ShapeDtypeStruct((NROW, C), jnp.float32),
        mesh=mesh,
        scratch_types=[
            pltpu.VMEM((2, 2, QB, 128), jnp.int32),
            pltpu.VMEM((2, 4, QB, 128), jnp.float32),
            pltpu.VMEM((2, 2 * QB * 128, 2 * D), jnp.bfloat16),
            pltpu.VMEM((QB, C), jnp.float32),
            pltpu.SemaphoreType.DMA,
            pltpu.SemaphoreType.DMA,
            pltpu.SemaphoreType.DMA,
            pltpu.SemaphoreType.DMA,
        ],
        compiler_params=pltpu.CompilerParams(
            use_tc_tiling_on_sc=False, needs_layout_passes=False),
    )
    return fn(table, *idxs, *wts)


def _permute_weights(w_off, b_off, w_attn, b_attn):
    perm_x, perm_y, perm_a = [], [], []
    for m in range(M):
        for l in range(L):
            for p in range(P):
                colb = ((m * L + l) * P + p) * 2
                perm_x.append(colb)
                perm_y.append(colb + 1)
                perm_a.append(m * 16 + p * 4 + l)
    perm_off = np.array(perm_x + perm_y, np.int32)
    perm_a = np.array(perm_a, np.int32)
    return (w_off[:, perm_off], b_off[perm_off],
            w_attn[:, perm_a], b_attn[perm_a])


def kernel(query, reference_points, input_flatten, input_spatial_shapes,
           W_off, b_off, W_attn, b_attn, W_val, b_val, W_out, b_out):
    f32 = jnp.float32
    wl_np, hl_np, base_np, sx_np, sy_np, bsum_np = _np_consts()
    w_offp, b_offp, w_attnp, b_attnp = _permute_weights(
        W_off, b_off, W_attn, b_attn)

    q2 = query.reshape(NROW, C)
    x2 = input_flatten.reshape(N * V, C)
    rpx = reference_points[..., 0].reshape(NROW, L)
    rpy = reference_points[..., 1].reshape(NROW, L)

    value, i0, i1, w0, w1, w2, w3 = _run_stage_a(
        q2, x2, rpx, rpy,
        W_val, b_val.reshape(1, 256),
        w_offp, b_offp.reshape(1, 256),
        w_attnp, b_attnp.reshape(1, 128),
        jnp.asarray(sx_np), jnp.asarray(sy_np),
        jnp.asarray(base_np).reshape(1, 128),
        jnp.asarray(wl_np).reshape(1, 128),
        jnp.asarray(hl_np).reshape(1, 128),
        jnp.asarray(bsum_np), jnp.asarray(bsum_np.T),
    )

    t = value.reshape(N, V, M, D).transpose(0, 2, 1, 3)
    t2 = jnp.concatenate([t, jnp.roll(t, -1, axis=2)], axis=3)
    table = t2.reshape(TABLE_ROWS, 2 * D)
    sampled = _run_stage_b(table, (i0, i1), (w0, w1, w2, w3))
    perm_rows = np.concatenate(
        [m * D + np.concatenate([np.arange(0, D, 2), np.arange(1, D, 2)])
         for m in range(M)]).astype(np.int32)
    out = _run_stage_c(sampled, W_out[perm_rows], b_out.reshape(1, 256))
    return out.reshape(N, LQ, C)


# speedup vs baseline: 1.2315x; 1.2315x over previous
import functools

import jax
import jax.numpy as jnp
import numpy as np
from jax import lax
from jax.experimental import pallas as pl
from jax.experimental.pallas import tpu as pltpu
from jax.experimental.pallas import tpu_sc as plsc

N, LQ, C = 4, 5440, 256
M, L, P, D = 8, 4, 4, 32
HW_LIST = [(64, 64), (32, 32), (16, 16), (8, 8)]
SIZES = [h * w for h, w in HW_LIST]
STARTS = [0, 4096, 5120, 5376]
V = 5440
NROW = N * LQ
QBLK = 320
GJ = LQ // QBLK
TABLE_ROWS = N * M * V

NC, NS, LANES = 2, 16, 16
NW = NC * NS
QSLICE = LQ // (NW // N)
QB = 4
ROWS_PER_CHUNK = QB * 4 * 128
NCHUNK = QSLICE // QB


def _np_consts():
    lvl = np.zeros(128, np.int32)
    for m in range(M):
        for l in range(L):
            for p in range(P):
                lvl[m * 16 + l * 4 + p] = l
    wl = np.array([HW_LIST[l][1] for l in lvl], np.float32)
    hl = np.array([HW_LIST[l][0] for l in lvl], np.float32)
    base = np.array(
        [STARTS[lvl[c]] * M + c // 16 for c in range(128)], np.float32
    )
    sx = np.zeros((4, 128), np.float32)
    sy = np.zeros((4, 128), np.float32)
    for col in range(128):
        sx[lvl[col], col] = wl[col]
        sy[lvl[col], col] = hl[col]
    bsum = np.zeros((128, 8), np.float32)
    for col in range(128):
        bsum[col, col // 16] = 1.0
    return wl, hl, base, sx, sy, bsum


def _stage_a_body(q_ref, x_ref, rpx_ref, rpy_ref, wv_ref, bv_ref, wo_ref,
                  bo_ref, wa_ref, ba_ref, sx_ref, sy_ref, base_ref, wl_ref,
                  hl_ref, bs_ref, bst_ref,
                  val_ref, i0_ref, i1_ref, i2_ref, i3_ref,
                  w0_ref, w1_ref, w2_ref, w3_ref):
    f32 = jnp.float32
    q = q_ref[...]
    val_ref[...] = (
        jnp.dot(x_ref[...], wv_ref[...], preferred_element_type=f32)
        + bv_ref[...]
    ).astype(jnp.bfloat16)
    off = jnp.dot(q, wo_ref[...], preferred_element_type=f32,
                precision=lax.Precision.HIGHEST) + bo_ref[...]
    lg = jnp.dot(q, wa_ref[...], preferred_element_type=f32,
                precision=lax.Precision.HIGHEST) + ba_ref[...]
    e = jnp.exp(lg)
    s = jnp.dot(e, bs_ref[...], preferred_element_type=f32,
                precision=lax.Precision.HIGHEST)
    rb = jnp.dot(1.0 / s, bst_ref[...], preferred_element_type=f32,
                precision=lax.Precision.HIGHEST)
    aw = e * rb
    offx = off[:, :128]
    offy = off[:, 128:]
    ix = jnp.dot(rpx_ref[...], sx_ref[...], preferred_element_type=f32,
                precision=lax.Precision.HIGHEST) \
        + offx - 0.5
    iy = jnp.dot(rpy_ref[...], sy_ref[...], preferred_element_type=f32,
                precision=lax.Precision.HIGHEST) \
        + offy - 0.5
    x0 = jnp.floor(ix)
    y0 = jnp.floor(iy)
    x1 = x0 + 1.0
    y1 = y0 + 1.0
    wx1 = ix - x0
    wx0 = 1.0 - wx1
    wy1 = iy - y0
    wy0 = 1.0 - wy1
    wl = wl_ref[...]
    hl = hl_ref[...]
    wm1 = wl - 1.0
    hm1 = hl - 1.0
    vx0 = ((x0 >= 0.0) & (x0 <= wm1)).astype(f32)
    vx1 = ((x1 >= 0.0) & (x1 <= wm1)).astype(f32)
    vy0 = ((y0 >= 0.0) & (y0 <= hm1)).astype(f32)
    vy1 = ((y1 >= 0.0) & (y1 <= hm1)).astype(f32)
    x0c = jnp.clip(x0, 0.0, wm1)
    x1c = jnp.clip(x1, 0.0, wm1)
    y0c = jnp.clip(y0, 0.0, hm1)
    y1c = jnp.clip(y1, 0.0, hm1)
    nb = pl.program_id(0).astype(f32)
    base = base_ref[...] + nb * jnp.float32(M * V)
    eight = jnp.float32(M)
    corners = [
        (i0_ref, w0_ref, x0c, wx0, vx0, y0c, wy0, vy0),
        (i1_ref, w1_ref, x1c, wx1, vx1, y0c, wy0, vy0),
        (i2_ref, w2_ref, x0c, wx0, vx0, y1c, wy1, vy1),
        (i3_ref, w3_ref, x1c, wx1, vx1, y1c, wy1, vy1),
    ]
    for iref, wref, xc, wxc, vxc, yc, wyc, vyc in corners:
        iref[...] = (base + (yc * wl + xc) * eight).astype(jnp.int32)
        wref[...] = aw * wxc * wyc * vxc * vyc


def _run_stage_a(q2, x2, rpx, rpy, w_val, b_val, w_offp, b_offp, w_attnp,
                 b_attnp, sx, sy, basev, wlv, hlv, bsum, bsum_t,
                 interpret=False):
    f32 = jnp.float32
    row_spec = lambda shp: pl.BlockSpec(
        (QBLK, shp), lambda n, j: (n * GJ + j, 0))
    full_spec = lambda a, b: pl.BlockSpec((a, b), lambda n, j: (0, 0))
    outs = [jax.ShapeDtypeStruct((NROW, 256), jnp.bfloat16)]
    outs += [jax.ShapeDtypeStruct((NROW, 128), jnp.int32)] * 4
    outs += [jax.ShapeDtypeStruct((NROW, 128), f32)] * 4
    return pl.pallas_call(
        _stage_a_body,
        grid=(N, GJ),
        in_specs=[
            row_spec(256), row_spec(256), row_spec(4), row_spec(4),
            full_spec(256, 256), full_spec(1, 256),
            full_spec(256, 256), full_spec(1, 256),
            full_spec(256, 128), full_spec(1, 128),
            full_spec(4, 128), full_spec(4, 128),
            full_spec(1, 128), full_spec(1, 128), full_spec(1, 128),
            full_spec(128, 8), full_spec(8, 128),
        ],
        out_specs=[row_spec(256)] + [row_spec(128)] * 8,
        out_shape=outs,
        interpret=interpret,
    )(q2, x2, rpx, rpy, w_val, b_val, w_offp, b_offp, w_attnp, b_attnp,
      sx, sy, basev, wlv, hlv, bsum, bsum_t)


def _matmul_body(x_ref, w_ref, b_ref, o_ref):
    o_ref[...] = (
        jnp.dot(x_ref[...], w_ref[...], preferred_element_type=jnp.float32)
        + b_ref[...]
    )


def _run_stage_c(x2, w_out, b_out, interpret=False):
    return pl.pallas_call(
        _matmul_body,
        grid=(N, GJ),
        in_specs=[
            pl.BlockSpec((QBLK, 256), lambda n, j: (n * GJ + j, 0)),
            pl.BlockSpec((256, 256), lambda n, j: (0, 0)),
            pl.BlockSpec((1, 256), lambda n, j: (0, 0)),
        ],
        out_specs=pl.BlockSpec((QBLK, 256), lambda n, j: (n * GJ + j, 0)),
        out_shape=jax.ShapeDtypeStruct((NROW, 256), jnp.float32),
        interpret=interpret,
    )(x2, w_out, b_out)


def _sc_body(table_ref, i0_ref, i1_ref, i2_ref, i3_ref,
             w0_ref, w1_ref, w2_ref, w3_ref, out_ref,
             idx_v, w_v, rows_v, out_v, g0, g1, iw0, iw1):
    g_sems = [g0, g1]
    iw_sems = [iw0, iw1]
    f32 = jnp.float32
    cid = lax.axis_index("c")
    sid = lax.axis_index("s")
    wid = sid * NC + cid
    n = wid // (NW // N)
    s = wid % (NW // N)
    q0 = s * QSLICE
    irefs = [i0_ref, i1_ref, i2_ref, i3_ref]
    wrefs = [w0_ref, w1_ref, w2_ref, w3_ref]

    def row0_of(tc):
        return n * LQ + q0 + tc * QB

    def issue_iw(tc, buf):
        r0 = row0_of(tc)
        for c in range(4):
            pltpu.async_copy(
                irefs[c].at[pl.ds(r0, QB)], idx_v.at[buf, c], iw_sems[buf])
            pltpu.async_copy(
                wrefs[c].at[pl.ds(r0, QB)], w_v.at[buf, c], iw_sems[buf])

    def wait_iw(buf):
        for c in range(4):
            pltpu.make_async_copy(
                irefs[c].at[pl.ds(0, QB)], idx_v.at[buf, c],
                iw_sems[buf]).wait()
            pltpu.make_async_copy(
                wrefs[c].at[pl.ds(0, QB)], w_v.at[buf, c],
                iw_sems[buf]).wait()

    def issue_gathers(buf):
        for c in range(4):
            for q in range(QB):
                pltpu.async_copy(
                    table_ref.at[idx_v.at[buf, c, q]],
                    rows_v.at[buf, pl.ds((c * QB + q) * 128, 128)],
                    g_sems[buf])

    def wait_gathers(buf):
        for c in range(4):
            for q in range(QB):
                pltpu.make_async_copy(
                    table_ref.at[idx_v.at[buf, c, q]],
                    rows_v.at[buf, pl.ds((c * QB + q) * 128, 128)],
                    g_sems[buf]).wait()

    def mac(tc, buf):
        def q_body(q, carry2):
            def m_body(m, carry3):
                p0 = [jnp.zeros((LANES,), f32) for _ in range(4)]
                p1 = [jnp.zeros((LANES,), f32) for _ in range(4)]
                w16s = [w_v[buf, c, q, pl.ds(m * 16, 16)] for c in range(4)]
                for r in range(16):
                    sel = jnp.full((LANES,), r, jnp.int32)
                    for c in range(4):
                        wb = w16s[c].at[sel].get(mode="promise_in_bounds")
                        row = (c * QB + q) * 128 + m * 16 + r
                        ra, rb = plsc.unpack(
                            rows_v[buf, row, pl.ds(0, 32)],
                            format=plsc.PackFormat.INTERLEAVED)
                        p0[c] = p0[c] + wb * ra
                        p1[c] = p1[c] + wb * rb
                out_v[q, pl.ds(m * D, 16)] = (p0[0] + p0[1]) + (p0[2] + p0[3])
                out_v[q, pl.ds(m * D + 16, 16)] = (
                    (p1[0] + p1[1]) + (p1[2] + p1[3]))
                return carry3

            lax.fori_loop(0, M, m_body, 0)
            return carry2

        lax.fori_loop(0, QB, q_body, 0)
        pltpu.sync_copy(out_v, out_ref.at[pl.ds(row0_of(tc), QB)])

    def half(t, buf):
        nbuf = 1 - buf
        wait_gathers(buf)
        wait_iw(nbuf)
        issue_gathers(nbuf)
        mac(t, buf)
        issue_iw(jnp.minimum(t + 2, NCHUNK - 1), buf)

    r0 = row0_of(0)
    for c in range(4):
        pltpu.sync_copy(irefs[c].at[pl.ds(r0, QB)], idx_v.at[0, c])
        pltpu.sync_copy(wrefs[c].at[pl.ds(r0, QB)], w_v.at[0, c])
    issue_gathers(0)
    issue_iw(jnp.int32(1), 1)

    def pair_body(i, carry):
        t0 = i * 2
        half(t0, 0)
        half(t0 + 1, 1)
        return carry

    lax.fori_loop(0, NCHUNK // 2, pair_body, 0)
    wait_gathers(0)
    wait_iw(1)


def _run_stage_b(table, idxs, wts):
    mesh = plsc.VectorSubcoreMesh(core_axis_name="c", subcore_axis_name="s")
    fn = pl.kernel(
        _sc_body,
        out_type=jax.ShapeDtypeStruct((NROW, C), jnp.float32),
        mesh=mesh,
        scratch_types=[
            pltpu.VMEM((2, 4, QB, 128), jnp.int32),
            pltpu.VMEM((2, 4, QB, 128), jnp.float32),
            pltpu.VMEM((2, ROWS_PER_CHUNK, D), jnp.bfloat16),
            pltpu.VMEM((QB, C), jnp.float32),
            pltpu.SemaphoreType.DMA,
            pltpu.SemaphoreType.DMA,
            pltpu.SemaphoreType.DMA,
            pltpu.SemaphoreType.DMA,
        ],
        compiler_params=pltpu.CompilerParams(
            use_tc_tiling_on_sc=False, needs_layout_passes=False),
    )
    return fn(table, *idxs, *wts)


def _permute_weights(w_off, b_off, w_attn, b_attn):
    perm_x, perm_y, perm_a = [], [], []
    for m in range(M):
        for l in range(L):
            for p in range(P):
                colb = ((m * L + l) * P + p) * 2
                perm_x.append(colb)
                perm_y.append(colb + 1)
                perm_a.append(m * 16 + p * 4 + l)
    perm_off = np.array(perm_x + perm_y, np.int32)
    perm_a = np.array(perm_a, np.int32)
    return (w_off[:, perm_off], b_off[perm_off],
            w_attn[:, perm_a], b_attn[perm_a])


def kernel(query, reference_points, input_flatten, input_spatial_shapes,
           W_off, b_off, W_attn, b_attn, W_val, b_val, W_out, b_out):
    f32 = jnp.float32
    wl_np, hl_np, base_np, sx_np, sy_np, bsum_np = _np_consts()
    w_offp, b_offp, w_attnp, b_attnp = _permute_weights(
        W_off, b_off, W_attn, b_attn)

    q2 = query.reshape(NROW, C)
    x2 = input_flatten.reshape(N * V, C)
    rpx = reference_points[..., 0].reshape(NROW, L)
    rpy = reference_points[..., 1].reshape(NROW, L)

    value, i0, i1, i2, i3, w0, w1, w2, w3 = _run_stage_a(
        q2, x2, rpx, rpy,
        W_val, b_val.reshape(1, 256),
        w_offp, b_offp.reshape(1, 256),
        w_attnp, b_attnp.reshape(1, 128),
        jnp.asarray(sx_np), jnp.asarray(sy_np),
        jnp.asarray(base_np).reshape(1, 128),
        jnp.asarray(wl_np).reshape(1, 128),
        jnp.asarray(hl_np).reshape(1, 128),
        jnp.asarray(bsum_np), jnp.asarray(bsum_np.T),
    )

    table = value.reshape(TABLE_ROWS, D)
    sampled = _run_stage_b(table, (i0, i1, i2, i3), (w0, w1, w2, w3))
    perm_rows = np.concatenate(
        [m * D + np.concatenate([np.arange(0, D, 2), np.arange(1, D, 2)])
         for m in range(M)]).astype(np.int32)
    out = _run_stage_c(sampled, W_out[perm_rows], b_out.reshape(1, 256))
    return out.reshape(N, LQ, C)


# speedup vs baseline: 1.2338x; 1.0019x over previous
import functools

import jax
import jax.numpy as jnp
import numpy as np
from jax import lax
from jax.experimental import pallas as pl
from jax.experimental.pallas import tpu as pltpu
from jax.experimental.pallas import tpu_sc as plsc

N, LQ, C = 4, 5440, 256
M, L, P, D = 8, 4, 4, 32
HW_LIST = [(64, 64), (32, 32), (16, 16), (8, 8)]
SIZES = [h * w for h, w in HW_LIST]
STARTS = [0, 4096, 5120, 5376]
V = 5440
NROW = N * LQ
QBLK = 320
GJ = LQ // QBLK
TABLE_ROWS = N * M * V

NC, NS, LANES = 2, 16, 16
NW = NC * NS
QSLICE = LQ // NS
QB = 4
ROWS_PER_CHUNK = QB * 4 * 128
NCHUNK = QSLICE // QB


def _np_consts():
    lvl = np.zeros(128, np.int32)
    for m in range(M):
        for l in range(L):
            for p in range(P):
                lvl[m * 16 + l * 4 + p] = l
    wl = np.array([HW_LIST[l][1] for l in lvl], np.float32)
    hl = np.array([HW_LIST[l][0] for l in lvl], np.float32)
    base = np.array(
        [STARTS[lvl[c]] * M + c // 16 for c in range(128)], np.float32
    )
    sx = np.zeros((4, 128), np.float32)
    sy = np.zeros((4, 128), np.float32)
    for col in range(128):
        sx[lvl[col], col] = wl[col]
        sy[lvl[col], col] = hl[col]
    bsum = np.zeros((128, 8), np.float32)
    for col in range(128):
        bsum[col, col // 16] = 1.0
    return wl, hl, base, sx, sy, bsum


def _stage_a_body(q_ref, x_ref, rpx_ref, rpy_ref, wv_ref, bv_ref, wo_ref,
                  bo_ref, wa_ref, ba_ref, sx_ref, sy_ref, base_ref, wl_ref,
                  hl_ref, bs_ref, bst_ref,
                  val_ref, i0_ref, i1_ref, i2_ref, i3_ref,
                  w0_ref, w1_ref, w2_ref, w3_ref):
    f32 = jnp.float32
    q = q_ref[...]
    val_ref[...] = (
        jnp.dot(x_ref[...], wv_ref[...], preferred_element_type=f32)
        + bv_ref[...]
    ).astype(jnp.bfloat16)
    off = jnp.dot(q, wo_ref[...], preferred_element_type=f32,
                precision=lax.Precision.HIGHEST) + bo_ref[...]
    lg = jnp.dot(q, wa_ref[...], preferred_element_type=f32,
                precision=lax.Precision.HIGHEST) + ba_ref[...]
    e = jnp.exp(lg)
    s = jnp.dot(e, bs_ref[...], preferred_element_type=f32,
                precision=lax.Precision.HIGHEST)
    rb = jnp.dot(1.0 / s, bst_ref[...], preferred_element_type=f32,
                precision=lax.Precision.HIGHEST)
    aw = e * rb
    offx = off[:, :128]
    offy = off[:, 128:]
    ix = jnp.dot(rpx_ref[...], sx_ref[...], preferred_element_type=f32,
                precision=lax.Precision.HIGHEST) \
        + offx - 0.5
    iy = jnp.dot(rpy_ref[...], sy_ref[...], preferred_element_type=f32,
                precision=lax.Precision.HIGHEST) \
        + offy - 0.5
    x0 = jnp.floor(ix)
    y0 = jnp.floor(iy)
    x1 = x0 + 1.0
    y1 = y0 + 1.0
    wx1 = ix - x0
    wx0 = 1.0 - wx1
    wy1 = iy - y0
    wy0 = 1.0 - wy1
    wl = wl_ref[...]
    hl = hl_ref[...]
    wm1 = wl - 1.0
    hm1 = hl - 1.0
    vx0 = ((x0 >= 0.0) & (x0 <= wm1)).astype(f32)
    vx1 = ((x1 >= 0.0) & (x1 <= wm1)).astype(f32)
    vy0 = ((y0 >= 0.0) & (y0 <= hm1)).astype(f32)
    vy1 = ((y1 >= 0.0) & (y1 <= hm1)).astype(f32)
    x0c = jnp.clip(x0, 0.0, wm1)
    x1c = jnp.clip(x1, 0.0, wm1)
    y0c = jnp.clip(y0, 0.0, hm1)
    y1c = jnp.clip(y1, 0.0, hm1)
    base = base_ref[...]
    eight = jnp.float32(M)
    corners = [
        (i0_ref, w0_ref, x0c, wx0, vx0, y0c, wy0, vy0),
        (i1_ref, w1_ref, x1c, wx1, vx1, y0c, wy0, vy0),
        (i2_ref, w2_ref, x0c, wx0, vx0, y1c, wy1, vy1),
        (i3_ref, w3_ref, x1c, wx1, vx1, y1c, wy1, vy1),
    ]
    for iref, wref, xc, wxc, vxc, yc, wyc, vyc in corners:
        iref[...] = (base + (yc * wl + xc) * eight).astype(jnp.int32)
        wref[...] = aw * wxc * wyc * vxc * vyc


def _run_stage_a(q2, x2, rpx, rpy, w_val, b_val, w_offp, b_offp, w_attnp,
                 b_attnp, sx, sy, basev, wlv, hlv, bsum, bsum_t,
                 interpret=False):
    f32 = jnp.float32
    row_spec = lambda shp: pl.BlockSpec(
        (QBLK, shp), lambda n, j: (n * GJ + j, 0))
    full_spec = lambda a, b: pl.BlockSpec((a, b), lambda n, j: (0, 0))
    outs = [jax.ShapeDtypeStruct((NROW, 256), jnp.bfloat16)]
    outs += [jax.ShapeDtypeStruct((NROW, 128), jnp.int32)] * 4
    outs += [jax.ShapeDtypeStruct((NROW, 128), f32)] * 4
    return pl.pallas_call(
        _stage_a_body,
        grid=(N, GJ),
        in_specs=[
            row_spec(256), row_spec(256), row_spec(4), row_spec(4),
            full_spec(256, 256), full_spec(1, 256),
            full_spec(256, 256), full_spec(1, 256),
            full_spec(256, 128), full_spec(1, 128),
            full_spec(4, 128), full_spec(4, 128),
            full_spec(1, 128), full_spec(1, 128), full_spec(1, 128),
            full_spec(128, 8), full_spec(8, 128),
        ],
        out_specs=[row_spec(256)] + [row_spec(128)] * 8,
        out_shape=outs,
        interpret=interpret,
    )(q2, x2, rpx, rpy, w_val, b_val, w_offp, b_offp, w_attnp, b_attnp,
      sx, sy, basev, wlv, hlv, bsum, bsum_t)


def _matmul_body(x_ref, w_ref, b_ref, o_ref):
    o_ref[...] = (
        jnp.dot(x_ref[...], w_ref[...], preferred_element_type=jnp.float32)
        + b_ref[...]
    )


def _run_stage_c(x2, w_out, b_out, interpret=False):
    return pl.pallas_call(
        _matmul_body,
        grid=(N, GJ),
        in_specs=[
            pl.BlockSpec((QBLK, 256), lambda n, j: (n * GJ + j, 0)),
            pl.BlockSpec((256, 256), lambda n, j: (0, 0)),
            pl.BlockSpec((1, 256), lambda n, j: (0, 0)),
        ],
        out_specs=pl.BlockSpec((QBLK, 256), lambda n, j: (n * GJ + j, 0)),
        out_shape=jax.ShapeDtypeStruct((NROW, 256), jnp.float32),
        interpret=interpret,
    )(x2, w_out, b_out)


def _sc_body(table_ref, i0_ref, i1_ref, i2_ref, i3_ref,
             w0_ref, w1_ref, w2_ref, w3_ref, out_ref,
             idx_v, w_v, rows_v, out_v, table_sp, g0, g1, iw0, iw1):
    g_sems = [g0, g1]
    iw_sems = [iw0, iw1]
    f32 = jnp.float32
    cid = lax.axis_index("c")
    sid = lax.axis_index("s")
    irefs = [i0_ref, i1_ref, i2_ref, i3_ref]
    wrefs = [w0_ref, w1_ref, w2_ref, w3_ref]
    q0 = sid * QSLICE
    phase_n = [2 * cid, 2 * cid + 1]
    nbox = [phase_n[0]]

    def row0_of(tc):
        return nbox[0] * LQ + q0 + tc * QB

    def issue_iw(tc, buf):
        r0 = row0_of(tc)
        for c in range(4):
            pltpu.async_copy(
                irefs[c].at[pl.ds(r0, QB)], idx_v.at[buf, c], iw_sems[buf])
            pltpu.async_copy(
                wrefs[c].at[pl.ds(r0, QB)], w_v.at[buf, c], iw_sems[buf])

    def wait_iw(buf):
        for c in range(4):
            pltpu.make_async_copy(
                irefs[c].at[pl.ds(0, QB)], idx_v.at[buf, c],
                iw_sems[buf]).wait()
            pltpu.make_async_copy(
                wrefs[c].at[pl.ds(0, QB)], w_v.at[buf, c],
                iw_sems[buf]).wait()

    def issue_gathers(buf):
        for c in range(4):
            for q in range(QB):
                pltpu.async_copy(
                    table_sp.at[idx_v.at[buf, c, q]],
                    rows_v.at[buf, pl.ds((c * QB + q) * 128, 128)],
                    g_sems[buf])

    def wait_gathers(buf):
        for c in range(4):
            for q in range(QB):
                pltpu.make_async_copy(
                    table_sp.at[idx_v.at[buf, c, q]],
                    rows_v.at[buf, pl.ds((c * QB + q) * 128, 128)],
                    g_sems[buf]).wait()

    def mac(tc, buf):
        def q_body(q, carry2):
            def m_body(m, carry3):
                p0 = [jnp.zeros((LANES,), f32) for _ in range(4)]
                p1 = [jnp.zeros((LANES,), f32) for _ in range(4)]
                w16s = [w_v[buf, c, q, pl.ds(m * 16, 16)] for c in range(4)]
                for r in range(16):
                    sel = jnp.full((LANES,), r, jnp.int32)
                    for c in range(4):
                        wb = w16s[c].at[sel].get(mode="promise_in_bounds")
                        row = (c * QB + q) * 128 + m * 16 + r
                        ra, rb = plsc.unpack(
                            rows_v[buf, row, pl.ds(0, 32)],
                            format=plsc.PackFormat.INTERLEAVED)
                        p0[c] = p0[c] + wb * ra
                        p1[c] = p1[c] + wb * rb
                out_v[q, pl.ds(m * D, 16)] = (p0[0] + p0[1]) + (p0[2] + p0[3])
                out_v[q, pl.ds(m * D + 16, 16)] = (
                    (p1[0] + p1[1]) + (p1[2] + p1[3]))
                return carry3

            lax.fori_loop(0, M, m_body, 0)
            return carry2

        lax.fori_loop(0, QB, q_body, 0)
        pltpu.sync_copy(out_v, out_ref.at[pl.ds(row0_of(tc), QB)])

    def half(t, buf):
        nbuf = 1 - buf
        wait_gathers(buf)
        wait_iw(nbuf)
        issue_gathers(nbuf)
        mac(t, buf)
        issue_iw(jnp.minimum(t + 2, NCHUNK - 1), buf)

    for phase in range(2):
        nbox[0] = phase_n[phase]
        if phase:
            plsc.subcore_barrier()
        seg = M * V // NS
        pltpu.sync_copy(
            table_ref.at[pl.ds(nbox[0] * (M * V) + sid * seg, seg)],
            table_sp.at[pl.ds(sid * seg, seg)])
        plsc.subcore_barrier()
        r0 = row0_of(0)
        for c in range(4):
            pltpu.sync_copy(irefs[c].at[pl.ds(r0, QB)], idx_v.at[0, c])
            pltpu.sync_copy(wrefs[c].at[pl.ds(r0, QB)], w_v.at[0, c])
        issue_gathers(0)
        issue_iw(jnp.int32(1), 1)

        def pair_body(i, carry):
            t0 = i * 2
            half(t0, 0)
            half(t0 + 1, 1)
            return carry

        lax.fori_loop(0, NCHUNK // 2, pair_body, 0)
        if NCHUNK % 2:
            wait_gathers(0)
            mac(jnp.int32(NCHUNK - 1), 0)
            wait_iw(1)
        else:
            wait_gathers(0)
            wait_iw(1)


def _run_stage_b(table, idxs, wts):
    mesh = plsc.VectorSubcoreMesh(core_axis_name="c", subcore_axis_name="s")
    fn = pl.kernel(
        _sc_body,
        out_type=jax.ShapeDtypeStruct((NROW, C), jnp.float32),
        mesh=mesh,
        scratch_types=[
            pltpu.VMEM((2, 4, QB, 128), jnp.int32),
            pltpu.VMEM((2, 4, QB, 128), jnp.float32),
            pltpu.VMEM((2, ROWS_PER_CHUNK, D), jnp.bfloat16),
            pltpu.VMEM((QB, C), jnp.float32),
            pltpu.VMEM_SHARED((M * V, D), jnp.bfloat16),
            pltpu.SemaphoreType.DMA,
            pltpu.SemaphoreType.DMA,
            pltpu.SemaphoreType.DMA,
            pltpu.SemaphoreType.DMA,
        ],
        compiler_params=pltpu.CompilerParams(
            use_tc_tiling_on_sc=False, needs_layout_passes=False),
    )
    return fn(table, *idxs, *wts)


def _permute_weights(w_off, b_off, w_attn, b_attn):
    perm_x, perm_y, perm_a = [], [], []
    for m in range(M):
        for l in range(L):
            for p in range(P):
                colb = ((m * L + l) * P + p) * 2
                perm_x.append(colb)
                perm_y.append(colb + 1)
                perm_a.append(m * 16 + p * 4 + l)
    perm_off = np.array(perm_x + perm_y, np.int32)
    perm_a = np.array(perm_a, np.int32)
    return (w_off[:, perm_off], b_off[perm_off],
            w_attn[:, perm_a], b_attn[perm_a])


def kernel(query, reference_points, input_flatten, input_spatial_shapes,
           W_off, b_off, W_attn, b_attn, W_val, b_val, W_out, b_out):
    f32 = jnp.float32
    wl_np, hl_np, base_np, sx_np, sy_np, bsum_np = _np_consts()
    w_offp, b_offp, w_attnp, b_attnp = _permute_weights(
        W_off, b_off, W_attn, b_attn)

    q2 = query.reshape(NROW, C)
    x2 = input_flatten.reshape(N * V, C)
    rpx = reference_points[..., 0].reshape(NROW, L)
    rpy = reference_points[..., 1].reshape(NROW, L)

    value, i0, i1, i2, i3, w0, w1, w2, w3 = _run_stage_a(
        q2, x2, rpx, rpy,
        W_val, b_val.reshape(1, 256),
        w_offp, b_offp.reshape(1, 256),
        w_attnp, b_attnp.reshape(1, 128),
        jnp.asarray(sx_np), jnp.asarray(sy_np),
        jnp.asarray(base_np).reshape(1, 128),
        jnp.asarray(wl_np).reshape(1, 128),
        jnp.asarray(hl_np).reshape(1, 128),
        jnp.asarray(bsum_np), jnp.asarray(bsum_np.T),
    )

    table = value.reshape(TABLE_ROWS, D)
    sampled = _run_stage_b(table, (i0, i1, i2, i3), (w0, w1, w2, w3))
    perm_rows = np.concatenate(
        [m * D + np.concatenate([np.arange(0, D, 2), np.arange(1, D, 2)])
         for m in range(M)]).astype(np.int32)
    out = _run_stage_c(sampled, W_out[perm_rows], b_out.reshape(1, 256))
    return out.reshape(N, LQ, C)


# speedup vs baseline: 1.2472x; 1.0108x over previous
import functools

import jax
import jax.numpy as jnp
import numpy as np
from jax import lax
from jax.experimental import pallas as pl
from jax.experimental.pallas import tpu as pltpu
from jax.experimental.pallas import tpu_sc as plsc

N, LQ, C = 4, 5440, 256
M, L, P, D = 8, 4, 4, 32
HW_LIST = [(64, 64), (32, 32), (16, 16), (8, 8)]
SIZES = [h * w for h, w in HW_LIST]
STARTS = [0, 4096, 5120, 5376]
V = 5440
NROW = N * LQ
QBLK = 320
GJ = LQ // QBLK
TABLE_ROWS = N * M * V

NC, NS, LANES = 2, 16, 16
NW = NC * NS
QSLICE = LQ // NS
QB = 4
ROWS_PER_CHUNK = QB * 4 * 128
NCHUNK = QSLICE // QB


def _np_consts():
    lvl = np.zeros(128, np.int32)
    for m in range(M):
        for l in range(L):
            for p in range(P):
                lvl[m * 16 + l * 4 + p] = l
    wl = np.array([HW_LIST[l][1] for l in lvl], np.float32)
    hl = np.array([HW_LIST[l][0] for l in lvl], np.float32)
    base = np.array(
        [STARTS[lvl[c]] * M + c // 16 for c in range(128)], np.float32
    )
    sx = np.zeros((4, 128), np.float32)
    sy = np.zeros((4, 128), np.float32)
    for col in range(128):
        sx[lvl[col], col] = wl[col]
        sy[lvl[col], col] = hl[col]
    bsum = np.zeros((128, 8), np.float32)
    for col in range(128):
        bsum[col, col // 16] = 1.0
    return wl, hl, base, sx, sy, bsum


def _stage_a_body(q_ref, x_ref, rpx_ref, rpy_ref, wv_ref, bv_ref, wo_ref,
                  wol_ref, bo_ref, wa_ref, wal_ref, ba_ref, sx_ref, sy_ref,
                  base_ref, wl_ref,
                  hl_ref, bs_ref, bst_ref,
                  val_ref, i0_ref, i1_ref, i2_ref, i3_ref,
                  w0_ref, w1_ref, w2_ref, w3_ref):
    f32 = jnp.float32
    q = q_ref[...]
    val_ref[...] = (
        jnp.dot(x_ref[...], wv_ref[...], preferred_element_type=f32)
        + bv_ref[...]
    ).astype(jnp.bfloat16)
    qhi = q.astype(jnp.bfloat16)
    qlo = (q - qhi.astype(f32)).astype(jnp.bfloat16)

    def dot3(whi_ref, wlo_ref):
        whi = whi_ref[...]
        return (jnp.dot(qhi, whi, preferred_element_type=f32)
                + jnp.dot(qhi, wlo_ref[...], preferred_element_type=f32)
                + jnp.dot(qlo, whi, preferred_element_type=f32))

    off = dot3(wo_ref, wol_ref) + bo_ref[...]
    lg = dot3(wa_ref, wal_ref) + ba_ref[...]
    e = jnp.exp(lg)
    s = jnp.dot(e, bs_ref[...], preferred_element_type=f32,
                precision=lax.Precision.HIGHEST)
    rb = jnp.dot(1.0 / s, bst_ref[...], preferred_element_type=f32,
                precision=lax.Precision.HIGHEST)
    aw = e * rb
    offx = off[:, :128]
    offy = off[:, 128:]
    ix = jnp.dot(rpx_ref[...], sx_ref[...], preferred_element_type=f32,
                precision=lax.Precision.HIGHEST) \
        + offx - 0.5
    iy = jnp.dot(rpy_ref[...], sy_ref[...], preferred_element_type=f32,
                precision=lax.Precision.HIGHEST) \
        + offy - 0.5
    x0 = jnp.floor(ix)
    y0 = jnp.floor(iy)
    x1 = x0 + 1.0
    y1 = y0 + 1.0
    wx1 = ix - x0
    wx0 = 1.0 - wx1
    wy1 = iy - y0
    wy0 = 1.0 - wy1
    wl = wl_ref[...]
    hl = hl_ref[...]
    wm1 = wl - 1.0
    hm1 = hl - 1.0
    vx0 = ((x0 >= 0.0) & (x0 <= wm1)).astype(f32)
    vx1 = ((x1 >= 0.0) & (x1 <= wm1)).astype(f32)
    vy0 = ((y0 >= 0.0) & (y0 <= hm1)).astype(f32)
    vy1 = ((y1 >= 0.0) & (y1 <= hm1)).astype(f32)
    x0c = jnp.clip(x0, 0.0, wm1)
    x1c = jnp.clip(x1, 0.0, wm1)
    y0c = jnp.clip(y0, 0.0, hm1)
    y1c = jnp.clip(y1, 0.0, hm1)
    base = base_ref[...]
    eight = jnp.float32(M)
    corners = [
        (i0_ref, w0_ref, x0c, wx0, vx0, y0c, wy0, vy0),
        (i1_ref, w1_ref, x1c, wx1, vx1, y0c, wy0, vy0),
        (i2_ref, w2_ref, x0c, wx0, vx0, y1c, wy1, vy1),
        (i3_ref, w3_ref, x1c, wx1, vx1, y1c, wy1, vy1),
    ]
    for iref, wref, xc, wxc, vxc, yc, wyc, vyc in corners:
        iref[...] = (base + (yc * wl + xc) * eight).astype(jnp.int32)
        wref[...] = aw * wxc * wyc * vxc * vyc


def _run_stage_a(q2, x2, rpx, rpy, w_val, b_val, w_offp, w_offp_lo, b_offp,
                 w_attnp, w_attnp_lo, b_attnp, sx, sy, basev, wlv, hlv,
                 bsum, bsum_t, interpret=False):
    f32 = jnp.float32
    row_spec = lambda shp: pl.BlockSpec(
        (QBLK, shp), lambda n, j: (n * GJ + j, 0))
    full_spec = lambda a, b: pl.BlockSpec((a, b), lambda n, j: (0, 0))
    outs = [jax.ShapeDtypeStruct((NROW, 256), jnp.bfloat16)]
    outs += [jax.ShapeDtypeStruct((NROW, 128), jnp.int32)] * 4
    outs += [jax.ShapeDtypeStruct((NROW, 128), f32)] * 4
    return pl.pallas_call(
        _stage_a_body,
        grid=(N, GJ),
        in_specs=[
            row_spec(256), row_spec(256), row_spec(4), row_spec(4),
            full_spec(256, 256), full_spec(1, 256),
            full_spec(256, 256), full_spec(256, 256), full_spec(1, 256),
            full_spec(256, 128), full_spec(256, 128), full_spec(1, 128),
            full_spec(4, 128), full_spec(4, 128),
            full_spec(1, 128), full_spec(1, 128), full_spec(1, 128),
            full_spec(128, 8), full_spec(8, 128),
        ],
        out_specs=[row_spec(256)] + [row_spec(128)] * 8,
        out_shape=outs,
        interpret=interpret,
    )(q2, x2, rpx, rpy, w_val, b_val, w_offp, w_offp_lo, b_offp,
      w_attnp, w_attnp_lo, b_attnp, sx, sy, basev, wlv, hlv, bsum, bsum_t)


def _matmul_body(x_ref, w_ref, b_ref, o_ref):
    o_ref[...] = (
        jnp.dot(x_ref[...], w_ref[...], preferred_element_type=jnp.float32)
        + b_ref[...]
    )


def _run_stage_c(x2, w_out, b_out, interpret=False):
    return pl.pallas_call(
        _matmul_body,
        grid=(N, GJ),
        in_specs=[
            pl.BlockSpec((QBLK, 256), lambda n, j: (n * GJ + j, 0)),
            pl.BlockSpec((256, 256), lambda n, j: (0, 0)),
            pl.BlockSpec((1, 256), lambda n, j: (0, 0)),
        ],
        out_specs=pl.BlockSpec((QBLK, 256), lambda n, j: (n * GJ + j, 0)),
        out_shape=jax.ShapeDtypeStruct((NROW, 256), jnp.float32),
        interpret=interpret,
    )(x2, w_out, b_out)


def _sc_body(table_ref, i0_ref, i1_ref, i2_ref, i3_ref,
             w0_ref, w1_ref, w2_ref, w3_ref, out_ref,
             idx_v, w_v, rows_v, out_v, table_sp, g0, g1, iw0, iw1):
    g_sems = [g0, g1]
    iw_sems = [iw0, iw1]
    f32 = jnp.float32
    cid = lax.axis_index("c")
    sid = lax.axis_index("s")
    irefs = [i0_ref, i1_ref, i2_ref, i3_ref]
    wrefs = [w0_ref, w1_ref, w2_ref, w3_ref]
    q0 = sid * QSLICE
    phase_n = [2 * cid, 2 * cid + 1]
    nbox = [phase_n[0]]

    def row0_of(tc):
        return nbox[0] * LQ + q0 + tc * QB

    def issue_iw(tc, buf):
        r0 = row0_of(tc)
        for c in range(4):
            pltpu.async_copy(
                irefs[c].at[pl.ds(r0, QB)], idx_v.at[buf, c], iw_sems[buf])
            pltpu.async_copy(
                wrefs[c].at[pl.ds(r0, QB)], w_v.at[buf, c], iw_sems[buf])

    def wait_iw(buf):
        for c in range(4):
            pltpu.make_async_copy(
                irefs[c].at[pl.ds(0, QB)], idx_v.at[buf, c],
                iw_sems[buf]).wait()
            pltpu.make_async_copy(
                wrefs[c].at[pl.ds(0, QB)], w_v.at[buf, c],
                iw_sems[buf]).wait()

    def issue_gathers(buf):
        for c in range(4):
            for q in range(QB):
                pltpu.async_copy(
                    table_sp.at[idx_v.at[buf, c, q]],
                    rows_v.at[buf, pl.ds((c * QB + q) * 128, 128)],
                    g_sems[buf])

    def wait_gathers(buf):
        for c in range(4):
            for q in range(QB):
                pltpu.make_async_copy(
                    table_sp.at[idx_v.at[buf, c, q]],
                    rows_v.at[buf, pl.ds((c * QB + q) * 128, 128)],
                    g_sems[buf]).wait()

    def mac(tc, buf):
        def q_body(q, carry2):
            def m_body(m, carry3):
                p0 = [jnp.zeros((LANES,), f32) for _ in range(4)]
                p1 = [jnp.zeros((LANES,), f32) for _ in range(4)]
                w16s = [w_v[buf, c, q, pl.ds(m * 16, 16)] for c in range(4)]
                for r in range(16):
                    sel = jnp.full((LANES,), r, jnp.int32)
                    for c in range(4):
                        wb = w16s[c].at[sel].get(mode="promise_in_bounds")
                        row = (c * QB + q) * 128 + m * 16 + r
                        ra, rb = plsc.unpack(
                            rows_v[buf, row, pl.ds(0, 32)],
                            format=plsc.PackFormat.INTERLEAVED)
                        p0[c] = p0[c] + wb * ra
                        p1[c] = p1[c] + wb * rb
                out_v[q, pl.ds(m * D, 16)] = (p0[0] + p0[1]) + (p0[2] + p0[3])
                out_v[q, pl.ds(m * D + 16, 16)] = (
                    (p1[0] + p1[1]) + (p1[2] + p1[3]))
                return carry3

            lax.fori_loop(0, M, m_body, 0)
            return carry2

        lax.fori_loop(0, QB, q_body, 0)
        pltpu.sync_copy(out_v, out_ref.at[pl.ds(row0_of(tc), QB)])

    def half(t, buf):
        nbuf = 1 - buf
        wait_gathers(buf)
        wait_iw(nbuf)
        issue_gathers(nbuf)
        mac(t, buf)
        issue_iw(jnp.minimum(t + 2, NCHUNK - 1), buf)

    for phase in range(2):
        nbox[0] = phase_n[phase]
        if phase:
            plsc.subcore_barrier()
        seg = M * V // NS
        pltpu.sync_copy(
            table_ref.at[pl.ds(nbox[0] * (M * V) + sid * seg, seg)],
            table_sp.at[pl.ds(sid * seg, seg)])
        plsc.subcore_barrier()
        r0 = row0_of(0)
        for c in range(4):
            pltpu.sync_copy(irefs[c].at[pl.ds(r0, QB)], idx_v.at[0, c])
            pltpu.sync_copy(wrefs[c].at[pl.ds(r0, QB)], w_v.at[0, c])
        issue_gathers(0)
        issue_iw(jnp.int32(1), 1)

        def pair_body(i, carry):
            t0 = i * 2
            half(t0, 0)
            half(t0 + 1, 1)
            return carry

        lax.fori_loop(0, NCHUNK // 2, pair_body, 0)
        if NCHUNK % 2:
            wait_gathers(0)
            mac(jnp.int32(NCHUNK - 1), 0)
            wait_iw(1)
        else:
            wait_gathers(0)
            wait_iw(1)


def _run_stage_b(table, idxs, wts):
    mesh = plsc.VectorSubcoreMesh(core_axis_name="c", subcore_axis_name="s")
    fn = pl.kernel(
        _sc_body,
        out_type=jax.ShapeDtypeStruct((NROW, C), jnp.float32),
        mesh=mesh,
        scratch_types=[
            pltpu.VMEM((2, 4, QB, 128), jnp.int32),
            pltpu.VMEM((2, 4, QB, 128), jnp.float32),
            pltpu.VMEM((2, ROWS_PER_CHUNK, D), jnp.bfloat16),
            pltpu.VMEM((QB, C), jnp.float32),
            pltpu.VMEM_SHARED((M * V, D), jnp.bfloat16),
            pltpu.SemaphoreType.DMA,
            pltpu.SemaphoreType.DMA,
            pltpu.SemaphoreType.DMA,
            pltpu.SemaphoreType.DMA,
        ],
        compiler_params=pltpu.CompilerParams(
            use_tc_tiling_on_sc=False, needs_layout_passes=False),
    )
    return fn(table, *idxs, *wts)


def _permute_weights(w_off, b_off, w_attn, b_attn):
    perm_x, perm_y, perm_a = [], [], []
    for m in range(M):
        for l in range(L):
            for p in range(P):
                colb = ((m * L + l) * P + p) * 2
                perm_x.append(colb)
                perm_y.append(colb + 1)
                perm_a.append(m * 16 + p * 4 + l)
    perm_off = np.array(perm_x + perm_y, np.int32)
    perm_a = np.array(perm_a, np.int32)
    return (w_off[:, perm_off], b_off[perm_off],
            w_attn[:, perm_a], b_attn[perm_a])


def kernel(query, reference_points, input_flatten, input_spatial_shapes,
           W_off, b_off, W_attn, b_attn, W_val, b_val, W_out, b_out):
    f32 = jnp.float32
    wl_np, hl_np, base_np, sx_np, sy_np, bsum_np = _np_consts()
    w_offp, b_offp, w_attnp, b_attnp = _permute_weights(
        W_off, b_off, W_attn, b_attn)

    q2 = query.reshape(NROW, C)
    x2 = input_flatten.reshape(N * V, C)
    rpx = reference_points[..., 0].reshape(NROW, L)
    rpy = reference_points[..., 1].reshape(NROW, L)

    w_offp_hi = w_offp.astype(jnp.bfloat16)
    w_offp_lo = (w_offp - w_offp_hi.astype(jnp.float32)).astype(jnp.bfloat16)
    w_attnp_hi = w_attnp.astype(jnp.bfloat16)
    w_attnp_lo = (w_attnp - w_attnp_hi.astype(jnp.float32)
                  ).astype(jnp.bfloat16)
    value, i0, i1, i2, i3, w0, w1, w2, w3 = _run_stage_a(
        q2, x2, rpx, rpy,
        W_val, b_val.reshape(1, 256),
        w_offp_hi, w_offp_lo, b_offp.reshape(1, 256),
        w_attnp_hi, w_attnp_lo, b_attnp.reshape(1, 128),
        jnp.asarray(sx_np), jnp.asarray(sy_np),
        jnp.asarray(base_np).reshape(1, 128),
        jnp.asarray(wl_np).reshape(1, 128),
        jnp.asarray(hl_np).reshape(1, 128),
        jnp.asarray(bsum_np), jnp.asarray(bsum_np.T),
    )

    table = value.reshape(TABLE_ROWS, D)
    sampled = _run_stage_b(table, (i0, i1, i2, i3), (w0, w1, w2, w3))
    perm_rows = np.concatenate(
        [m * D + np.concatenate([np.arange(0, D, 2), np.arange(1, D, 2)])
         for m in range(M)]).astype(np.int32)
    out = _run_stage_c(sampled, W_out[perm_rows], b_out.reshape(1, 256))
    return out.reshape(N, LQ, C)


# speedup vs baseline: 1.5479x; 1.2411x over previous
import functools

import jax
import jax.numpy as jnp
import numpy as np
from jax import lax
from jax.experimental import pallas as pl
from jax.experimental.pallas import tpu as pltpu
from jax.experimental.pallas import tpu_sc as plsc

N, LQ, C = 4, 5440, 256
M, L, P, D = 8, 4, 4, 32
HW_LIST = [(64, 64), (32, 32), (16, 16), (8, 8)]
SIZES = [h * w for h, w in HW_LIST]
STARTS = [0, 4096, 5120, 5376]
V = 5440
NROW = N * LQ
QBLK = 320
GJ = LQ // QBLK
TABLE_ROWS = N * M * V

NC, NS, LANES = 2, 16, 16
NW = NC * NS
QSLICE = LQ // NS
QB = 4
ROWS_PER_CHUNK = QB * 4 * 128
NCHUNK = QSLICE // QB


def _np_consts():
    lvl = np.zeros(128, np.int32)
    for m in range(M):
        for l in range(L):
            for p in range(P):
                lvl[m * 16 + l * 4 + p] = l
    wl = np.array([HW_LIST[l][1] for l in lvl], np.float32)
    hl = np.array([HW_LIST[l][0] for l in lvl], np.float32)
    base = np.array(
        [STARTS[lvl[c]] * M + c // 16 for c in range(128)], np.float32
    )
    sx = np.zeros((4, 128), np.float32)
    sy = np.zeros((4, 128), np.float32)
    for col in range(128):
        sx[lvl[col], col] = wl[col]
        sy[lvl[col], col] = hl[col]
    bsum = np.zeros((128, 8), np.float32)
    for col in range(128):
        bsum[col, col // 16] = 1.0
    return wl, hl, base, sx, sy, bsum


def _stage_a_body(q_ref, x_ref, rpx_ref, rpy_ref, wv_ref, bv_ref, wo_ref,
                  wol_ref, bo_ref, wa_ref, wal_ref, ba_ref, sx_ref, sy_ref,
                  base_ref, wl_ref,
                  hl_ref, bs_ref, bst_ref,
                  val_ref, i0_ref, i1_ref, i2_ref, i3_ref,
                  w0_ref, w1_ref, w2_ref, w3_ref):
    f32 = jnp.float32
    q = q_ref[...]
    val_ref[...] = (
        jnp.dot(x_ref[...], wv_ref[...], preferred_element_type=f32)
        + bv_ref[...]
    ).astype(jnp.bfloat16)
    qhi = q.astype(jnp.bfloat16)
    qlo = (q - qhi.astype(f32)).astype(jnp.bfloat16)

    def dot3(whi_ref, wlo_ref):
        whi = whi_ref[...]
        return (jnp.dot(qhi, whi, preferred_element_type=f32)
                + jnp.dot(qhi, wlo_ref[...], preferred_element_type=f32)
                + jnp.dot(qlo, whi, preferred_element_type=f32))

    off = dot3(wo_ref, wol_ref) + bo_ref[...]
    lg = dot3(wa_ref, wal_ref) + ba_ref[...]
    e = jnp.exp(lg)
    s = jnp.dot(e, bs_ref[...], preferred_element_type=f32,
                precision=lax.Precision.HIGHEST)
    rb = jnp.dot(1.0 / s, bst_ref[...], preferred_element_type=f32,
                precision=lax.Precision.HIGHEST)
    aw = e * rb
    offx = off[:, :128]
    offy = off[:, 128:]
    ix = jnp.dot(rpx_ref[...], sx_ref[...], preferred_element_type=f32,
                precision=lax.Precision.HIGHEST) \
        + offx - 0.5
    iy = jnp.dot(rpy_ref[...], sy_ref[...], preferred_element_type=f32,
                precision=lax.Precision.HIGHEST) \
        + offy - 0.5
    x0 = jnp.floor(ix)
    y0 = jnp.floor(iy)
    x1 = x0 + 1.0
    y1 = y0 + 1.0
    wx1 = ix - x0
    wx0 = 1.0 - wx1
    wy1 = iy - y0
    wy0 = 1.0 - wy1
    wl = wl_ref[...]
    hl = hl_ref[...]
    wm1 = wl - 1.0
    hm1 = hl - 1.0
    vx0 = ((x0 >= 0.0) & (x0 <= wm1)).astype(f32)
    vx1 = ((x1 >= 0.0) & (x1 <= wm1)).astype(f32)
    vy0 = ((y0 >= 0.0) & (y0 <= hm1)).astype(f32)
    vy1 = ((y1 >= 0.0) & (y1 <= hm1)).astype(f32)
    x0c = jnp.clip(x0, 0.0, wm1)
    x1c = jnp.clip(x1, 0.0, wm1)
    y0c = jnp.clip(y0, 0.0, hm1)
    y1c = jnp.clip(y1, 0.0, hm1)
    base = base_ref[...]
    eight = jnp.float32(M)
    corners = [
        (i0_ref, w0_ref, x0c, wx0, vx0, y0c, wy0, vy0),
        (i1_ref, w1_ref, x1c, wx1, vx1, y0c, wy0, vy0),
        (i2_ref, w2_ref, x0c, wx0, vx0, y1c, wy1, vy1),
        (i3_ref, w3_ref, x1c, wx1, vx1, y1c, wy1, vy1),
    ]
    for iref, wref, xc, wxc, vxc, yc, wyc, vyc in corners:
        iref[...] = (base + (yc * wl + xc) * eight).astype(jnp.int32)
        wref[...] = aw * wxc * wyc * vxc * vyc


def _run_stage_a(q2, x2, rpx, rpy, w_val, b_val, w_offp, w_offp_lo, b_offp,
                 w_attnp, w_attnp_lo, b_attnp, sx, sy, basev, wlv, hlv,
                 bsum, bsum_t, interpret=False):
    f32 = jnp.float32
    row_spec = lambda shp: pl.BlockSpec(
        (QBLK, shp), lambda n, j: (n * GJ + j, 0))
    full_spec = lambda a, b: pl.BlockSpec((a, b), lambda n, j: (0, 0))
    outs = [jax.ShapeDtypeStruct((NROW, 256), jnp.bfloat16)]
    outs += [jax.ShapeDtypeStruct((NROW, 128), jnp.int32)] * 4
    outs += [jax.ShapeDtypeStruct((NROW, 128), f32)] * 4
    return pl.pallas_call(
        _stage_a_body,
        grid=(N, GJ),
        in_specs=[
            row_spec(256), row_spec(256), row_spec(4), row_spec(4),
            full_spec(256, 256), full_spec(1, 256),
            full_spec(256, 256), full_spec(256, 256), full_spec(1, 256),
            full_spec(256, 128), full_spec(256, 128), full_spec(1, 128),
            full_spec(4, 128), full_spec(4, 128),
            full_spec(1, 128), full_spec(1, 128), full_spec(1, 128),
            full_spec(128, 8), full_spec(8, 128),
        ],
        out_specs=[row_spec(256)] + [row_spec(128)] * 8,
        out_shape=outs,
        interpret=interpret,
    )(q2, x2, rpx, rpy, w_val, b_val, w_offp, w_offp_lo, b_offp,
      w_attnp, w_attnp_lo, b_attnp, sx, sy, basev, wlv, hlv, bsum, bsum_t)


def _matmul_body(x_ref, w_ref, b_ref, o_ref):
    o_ref[...] = (
        jnp.dot(x_ref[...], w_ref[...], preferred_element_type=jnp.float32)
        + b_ref[...]
    )


def _run_stage_c(x2, w_out, b_out, interpret=False):
    return pl.pallas_call(
        _matmul_body,
        grid=(N, GJ),
        in_specs=[
            pl.BlockSpec((QBLK, 256), lambda n, j: (n * GJ + j, 0)),
            pl.BlockSpec((256, 256), lambda n, j: (0, 0)),
            pl.BlockSpec((1, 256), lambda n, j: (0, 0)),
        ],
        out_specs=pl.BlockSpec((QBLK, 256), lambda n, j: (n * GJ + j, 0)),
        out_shape=jax.ShapeDtypeStruct((NROW, 256), jnp.float32),
        interpret=interpret,
    )(x2, w_out, b_out)


def _sc_body(table_ref, i0_ref, i1_ref, i2_ref, i3_ref,
             w0_ref, w1_ref, w2_ref, w3_ref, out_ref,
             idx_v, w_v, rows_v, out_v, table_sp, g0, g1, iw0, iw1):
    g_sems = [g0, g1]
    iw_sems = [iw0, iw1]
    f32 = jnp.float32
    cid = lax.axis_index("c")
    sid = lax.axis_index("s")
    irefs = [i0_ref, i1_ref, i2_ref, i3_ref]
    wrefs = [w0_ref, w1_ref, w2_ref, w3_ref]
    q0 = sid * QSLICE
    phase_n = [2 * cid, 2 * cid + 1]
    nbox = [phase_n[0]]

    def row0_of(tc):
        return nbox[0] * LQ + q0 + tc * QB

    def issue_iw(tc, buf):
        r0 = row0_of(tc)
        for c in range(4):
            pltpu.async_copy(
                irefs[c].at[pl.ds(r0, QB)], idx_v.at[buf, c], iw_sems[buf])
            pltpu.async_copy(
                wrefs[c].at[pl.ds(r0, QB)], w_v.at[buf, c], iw_sems[buf])

    def wait_iw(buf):
        for c in range(4):
            pltpu.make_async_copy(
                irefs[c].at[pl.ds(0, QB)], idx_v.at[buf, c],
                iw_sems[buf]).wait()
            pltpu.make_async_copy(
                wrefs[c].at[pl.ds(0, QB)], w_v.at[buf, c],
                iw_sems[buf]).wait()

    def issue_gathers(buf):
        for c in range(4):
            for q in range(QB):
                pltpu.async_copy(
                    table_sp.at[idx_v.at[buf, c, q]],
                    rows_v.at[buf, pl.ds((c * QB + q) * 128, 128)],
                    g_sems[buf])

    def wait_gathers(buf):
        for c in range(4):
            for q in range(QB):
                pltpu.make_async_copy(
                    table_sp.at[idx_v.at[buf, c, q]],
                    rows_v.at[buf, pl.ds((c * QB + q) * 128, 128)],
                    g_sems[buf]).wait()

    def mac(tc, buf):
        def q_body(q, carry2):
            def m_body(m, carry3):
                pc = [jnp.zeros((2 * LANES,), jnp.bfloat16) for _ in range(4)]
                w16s = [w_v[buf, c, q, pl.ds(m * 16, 16)] for c in range(4)]
                for r in range(16):
                    sel = jnp.full((LANES,), r, jnp.int32)
                    for c in range(4):
                        wb = w16s[c].at[sel].get(mode="promise_in_bounds")
                        wb2 = plsc.pack(
                            wb, wb, format=plsc.PackFormat.INTERLEAVED)
                        row = (c * QB + q) * 128 + m * 16 + r
                        pc[c] = pc[c] + wb2 * rows_v[buf, row, pl.ds(0, 32)]
                up = [plsc.unpack(p, format=plsc.PackFormat.INTERLEAVED)
                      for p in pc]
                out_v[q, pl.ds(m * D, 16)] = (
                    (up[0][0] + up[1][0]) + (up[2][0] + up[3][0]))
                out_v[q, pl.ds(m * D + 16, 16)] = (
                    (up[0][1] + up[1][1]) + (up[2][1] + up[3][1]))
                return carry3

            lax.fori_loop(0, M, m_body, 0)
            return carry2

        lax.fori_loop(0, QB, q_body, 0)
        pltpu.sync_copy(out_v, out_ref.at[pl.ds(row0_of(tc), QB)])

    def half(t, buf):
        nbuf = 1 - buf
        wait_gathers(buf)
        wait_iw(nbuf)
        issue_gathers(nbuf)
        mac(t, buf)
        issue_iw(jnp.minimum(t + 2, NCHUNK - 1), buf)

    for phase in range(2):
        nbox[0] = phase_n[phase]
        if phase:
            plsc.subcore_barrier()
        seg = M * V // NS
        pltpu.sync_copy(
            table_ref.at[pl.ds(nbox[0] * (M * V) + sid * seg, seg)],
            table_sp.at[pl.ds(sid * seg, seg)])
        plsc.subcore_barrier()
        r0 = row0_of(0)
        for c in range(4):
            pltpu.sync_copy(irefs[c].at[pl.ds(r0, QB)], idx_v.at[0, c])
            pltpu.sync_copy(wrefs[c].at[pl.ds(r0, QB)], w_v.at[0, c])
        issue_gathers(0)
        issue_iw(jnp.int32(1), 1)

        def pair_body(i, carry):
            t0 = i * 2
            half(t0, 0)
            half(t0 + 1, 1)
            return carry

        lax.fori_loop(0, NCHUNK // 2, pair_body, 0)
        if NCHUNK % 2:
            wait_gathers(0)
            mac(jnp.int32(NCHUNK - 1), 0)
            wait_iw(1)
        else:
            wait_gathers(0)
            wait_iw(1)


def _run_stage_b(table, idxs, wts):
    mesh = plsc.VectorSubcoreMesh(core_axis_name="c", subcore_axis_name="s")
    fn = pl.kernel(
        _sc_body,
        out_type=jax.ShapeDtypeStruct((NROW, C), jnp.float32),
        mesh=mesh,
        scratch_types=[
            pltpu.VMEM((2, 4, QB, 128), jnp.int32),
            pltpu.VMEM((2, 4, QB, 128), jnp.float32),
            pltpu.VMEM((2, ROWS_PER_CHUNK, D), jnp.bfloat16),
            pltpu.VMEM((QB, C), jnp.float32),
            pltpu.VMEM_SHARED((M * V, D), jnp.bfloat16),
            pltpu.SemaphoreType.DMA,
            pltpu.SemaphoreType.DMA,
            pltpu.SemaphoreType.DMA,
            pltpu.SemaphoreType.DMA,
        ],
        compiler_params=pltpu.CompilerParams(
            use_tc_tiling_on_sc=False, needs_layout_passes=False),
    )
    return fn(table, *idxs, *wts)


def _permute_weights(w_off, b_off, w_attn, b_attn):
    perm_x, perm_y, perm_a = [], [], []
    for m in range(M):
        for l in range(L):
            for p in range(P):
                colb = ((m * L + l) * P + p) * 2
                perm_x.append(colb)
                perm_y.append(colb + 1)
                perm_a.append(m * 16 + p * 4 + l)
    perm_off = np.array(perm_x + perm_y, np.int32)
    perm_a = np.array(perm_a, np.int32)
    return (w_off[:, perm_off], b_off[perm_off],
            w_attn[:, perm_a], b_attn[perm_a])


def kernel(query, reference_points, input_flatten, input_spatial_shapes,
           W_off, b_off, W_attn, b_attn, W_val, b_val, W_out, b_out):
    f32 = jnp.float32
    wl_np, hl_np, base_np, sx_np, sy_np, bsum_np = _np_consts()
    w_offp, b_offp, w_attnp, b_attnp = _permute_weights(
        W_off, b_off, W_attn, b_attn)

    q2 = query.reshape(NROW, C)
    x2 = input_flatten.reshape(N * V, C)
    rpx = reference_points[..., 0].reshape(NROW, L)
    rpy = reference_points[..., 1].reshape(NROW, L)

    w_offp_hi = w_offp.astype(jnp.bfloat16)
    w_offp_lo = (w_offp - w_offp_hi.astype(jnp.float32)).astype(jnp.bfloat16)
    w_attnp_hi = w_attnp.astype(jnp.bfloat16)
    w_attnp_lo = (w_attnp - w_attnp_hi.astype(jnp.float32)
                  ).astype(jnp.bfloat16)
    value, i0, i1, i2, i3, w0, w1, w2, w3 = _run_stage_a(
        q2, x2, rpx, rpy,
        W_val, b_val.reshape(1, 256),
        w_offp_hi, w_offp_lo, b_offp.reshape(1, 256),
        w_attnp_hi, w_attnp_lo, b_attnp.reshape(1, 128),
        jnp.asarray(sx_np), jnp.asarray(sy_np),
        jnp.asarray(base_np).reshape(1, 128),
        jnp.asarray(wl_np).reshape(1, 128),
        jnp.asarray(hl_np).reshape(1, 128),
        jnp.asarray(bsum_np), jnp.asarray(bsum_np.T),
    )

    table = value.reshape(TABLE_ROWS, D)
    sampled = _run_stage_b(table, (i0, i1, i2, i3), (w0, w1, w2, w3))
    perm_rows = np.concatenate(
        [m * D + np.concatenate([np.arange(0, D, 2), np.arange(1, D, 2)])
         for m in range(M)]).astype(np.int32)
    out = _run_stage_c(sampled, W_out[perm_rows], b_out.reshape(1, 256))
    return out.reshape(N, LQ, C)
